# baseline (speedup 1.0000x reference)
import math

import jax
import jax.numpy as jnp
from jax.experimental import pallas as pl
from jax.experimental.pallas import tpu as pltpu

_BF16 = jnp.bfloat16
_N_LAYERS = 4
_N_HEADS = 8


def _bf(a):
    return a.astype(_BF16)


def _layernorm(y, g, b):
    mu = jnp.mean(y, axis=-1, keepdims=True)
    var = jnp.mean((y - mu) ** 2, axis=-1, keepdims=True)
    return (y - mu) * jax.lax.rsqrt(var + 1e-5) * g + b


def _make_fused_kernel(S, D, H, d_fc, n_batch):
    dh = D // H
    scale = 1.0 / math.sqrt(dh)
    M = n_batch * S

    def _body(x_ref, wqkv_ref, bqkv_ref, wo_ref, bo_ref, ln1g_ref, ln1b_ref,
              w1_ref, b1_ref, w2_ref, b2_ref, ln2g_ref, ln2b_ref,
              mlm_w_ref, mlm_b_ref, o_mlm_ref, o_x_ref, xbuf):
        l = pl.program_id(1)

        @pl.when(l == 0)
        def _():
            xbuf[...] = x_ref[...].reshape(M, D)

        x = xbuf[...]

        qkv = (jnp.dot(_bf(x), wqkv_ref[0],
                       preferred_element_type=jnp.float32) + bqkv_ref[0])

        scores = []
        vhs = []
        for b in range(n_batch):
            rows = slice(b * S, (b + 1) * S)
            for h in range(H):
                qh = qkv[rows, h * dh:(h + 1) * dh] * scale
                kh = qkv[rows, D + h * dh:D + (h + 1) * dh]
                vhs.append(qkv[rows, 2 * D + h * dh:2 * D + (h + 1) * dh])
                scores.append(jax.lax.dot_general(
                    _bf(qh), _bf(kh), (((1,), (1,)), ((), ())),
                    preferred_element_type=jnp.float32))
        outs = []
        prev_p = None
        for i, s in enumerate(scores):
            p = jnp.exp(s - jnp.max(s, axis=-1, keepdims=True))
            p = p * pl.reciprocal(jnp.sum(p, axis=-1, keepdims=True),
                                  approx=True)
            if prev_p is not None:
                outs.append(jnp.dot(_bf(prev_p), _bf(vhs[i - 1]),
                                    preferred_element_type=jnp.float32))
            prev_p = p
        outs.append(jnp.dot(_bf(prev_p), _bf(vhs[-1]),
                            preferred_element_type=jnp.float32))
        attn = jnp.concatenate(
            [jnp.concatenate(outs[b * H:(b + 1) * H], axis=-1)
             for b in range(n_batch)], axis=0)

        proj = (jnp.dot(_bf(attn), wo_ref[0],
                        preferred_element_type=jnp.float32) + bo_ref[0])
        x1 = _layernorm(proj + x, ln1g_ref[0], ln1b_ref[0])

        hdn = jnp.maximum(
            jnp.dot(_bf(x1), w1_ref[0],
                    preferred_element_type=jnp.float32) + b1_ref[0], 0.0)
        y2 = (jnp.dot(_bf(hdn), w2_ref[0],
                      preferred_element_type=jnp.float32) + b2_ref[0]) + x1
        xn = _layernorm(y2, ln2g_ref[0], ln2b_ref[0])
        xbuf[...] = xn

        @pl.when(l == _N_LAYERS - 1)
        def _():
            o_x_ref[...] = xn.reshape(n_batch, S, D)
            logits = (jnp.dot(_bf(xn), mlm_w_ref[...],
                              preferred_element_type=jnp.float32)
                      + mlm_b_ref[...])
            o_mlm_ref[...] = logits.reshape(n_batch, S, -1)

    return _body


def _encoder_and_mlm(x, stacked, mlm_w, mlm_b, n_heads):
    B, S, D = x.shape
    d_fc = stacked[7].shape[2]
    Npad = mlm_w.shape[1]
    n_prog = 2
    nb = B // n_prog

    def lspec(shape):
        return pl.BlockSpec((1,) + shape[1:], lambda b, l: (l,) + (0,) * (len(shape) - 1))

    def cspec(shape):
        nd = len(shape)
        return pl.BlockSpec(shape, lambda b, l: (0,) * nd)

    in_specs = [pl.BlockSpec((nb, S, D), lambda b, l: (b, 0, 0))]
    in_specs += [lspec(a.shape) for a in stacked]
    in_specs += [cspec(mlm_w.shape), cspec(mlm_b.shape)]

    return pl.pallas_call(
        _make_fused_kernel(S, D, n_heads, d_fc, nb),
        out_shape=[jax.ShapeDtypeStruct((B, S, Npad), jnp.float32),
                   jax.ShapeDtypeStruct((B, S, D), jnp.float32)],
        grid=(n_prog, _N_LAYERS),
        in_specs=in_specs,
        out_specs=[pl.BlockSpec((nb, S, Npad), lambda b, l: (b, 0, 0)),
                   pl.BlockSpec((nb, S, D), lambda b, l: (b, 0, 0))],
        scratch_shapes=[pltpu.VMEM((nb * S, D), jnp.float32)],
        compiler_params=pltpu.CompilerParams(
            dimension_semantics=("parallel", "arbitrary")),
    )(x, *stacked, mlm_w, mlm_b)


def kernel(x_ids, token_emb, pos_emb, mlm_w_pad, mlm_b_pad, cls_w, cls_b,
           l0_wqkv, l0_bqkv, l0_wo, l0_bo, l0_w1, l0_b1, l0_w2, l0_b2,
           l0_ln1_g, l0_ln1_b, l0_ln2_g, l0_ln2_b,
           l1_wqkv, l1_bqkv, l1_wo, l1_bo, l1_w1, l1_b1, l1_w2, l1_b2,
           l1_ln1_g, l1_ln1_b, l1_ln2_g, l1_ln2_b,
           l2_wqkv, l2_bqkv, l2_wo, l2_bo, l2_w1, l2_b1, l2_w2, l2_b2,
           l2_ln1_g, l2_ln1_b, l2_ln2_g, l2_ln2_b,
           l3_wqkv, l3_bqkv, l3_wo, l3_bo, l3_w1, l3_b1, l3_w2, l3_b2,
           l3_ln1_g, l3_ln1_b, l3_ln2_g, l3_ln2_b):
    B, S = x_ids.shape
    mlm_cls_num = token_emb.shape[0]

    x = jnp.take(token_emb, x_ids, axis=0) + pos_emb[:S][None, :, :]

    stacked = [
        _bf(jnp.stack([l0_wqkv, l1_wqkv, l2_wqkv, l3_wqkv])),
        jnp.stack([l0_bqkv, l1_bqkv, l2_bqkv, l3_bqkv]),
        _bf(jnp.stack([l0_wo, l1_wo, l2_wo, l3_wo])),
        jnp.stack([l0_bo, l1_bo, l2_bo, l3_bo]),
        jnp.stack([l0_ln1_g, l1_ln1_g, l2_ln1_g, l3_ln1_g]),
        jnp.stack([l0_ln1_b, l1_ln1_b, l2_ln1_b, l3_ln1_b]),
        _bf(jnp.stack([l0_w1, l1_w1, l2_w1, l3_w1])),
        jnp.stack([l0_b1, l1_b1, l2_b1, l3_b1]),
        _bf(jnp.stack([l0_w2, l1_w2, l2_w2, l3_w2])),
        jnp.stack([l0_b2, l1_b2, l2_b2, l3_b2]),
        jnp.stack([l0_ln2_g, l1_ln2_g, l2_ln2_g, l3_ln2_g]),
        jnp.stack([l0_ln2_b, l1_ln2_b, l2_ln2_b, l3_ln2_b]),
    ]

    logits_pad, x_out = _encoder_and_mlm(
        x, stacked, _bf(mlm_w_pad), mlm_b_pad, _N_HEADS)

    yp_mlm = logits_pad[:, :, :mlm_cls_num]
    yp_cls = jnp.dot(x_out[:, 0, :], cls_w) + cls_b
    return yp_mlm, yp_cls


# speedup vs baseline: 1.3926x; 1.3926x over previous
import math

import jax
import jax.numpy as jnp
from jax.experimental import pallas as pl
from jax.experimental.pallas import tpu as pltpu

_BF16 = jnp.bfloat16
_N_LAYERS = 4
_N_HEADS = 8


def _bf(a):
    return a.astype(_BF16)


def _layernorm(y, g, b):
    mu = jnp.mean(y, axis=-1, keepdims=True)
    var = jnp.mean((y - mu) ** 2, axis=-1, keepdims=True)
    return (y - mu) * jax.lax.rsqrt(var + 1e-5) * g + b


def _make_fused_kernel(S, D, H, d_fc, n_batch):
    dh = D // H
    scale = 1.0 / math.sqrt(dh)
    M = n_batch * S

    def _body(*refs):
        x_ref = refs[0]
        lrefs = refs[1:1 + 12 * _N_LAYERS]
        mlm_w_hbm = refs[1 + 12 * _N_LAYERS]
        mlm_b_ref = refs[2 + 12 * _N_LAYERS]
        o_mlm_ref = refs[3 + 12 * _N_LAYERS]
        o_x_ref = refs[4 + 12 * _N_LAYERS]
        (wq_buf, wo_buf, w1_buf, w2_buf, mbuf, sems, msem) = refs[
            5 + 12 * _N_LAYERS:]

        bufs = (wq_buf, wo_buf, w1_buf, w2_buf)

        def layer_copies(l):
            slot = l % 2
            lw = lrefs[12 * l:12 * (l + 1)]
            hbm = (lw[0], lw[2], lw[6], lw[8])
            return [pltpu.make_async_copy(hbm[t], bufs[t].at[slot],
                                          sems.at[t, slot])
                    for t in range(4)]

        def start_layer(l):
            for c in layer_copies(l):
                c.start()

        def wait_layer(l):
            for c in layer_copies(l):
                c.wait()

        start_layer(0)
        start_layer(1)
        mlm_copy = pltpu.make_async_copy(mlm_w_hbm, mbuf, msem)

        x = x_ref[...].reshape(M, D)
        for l in range(_N_LAYERS):
            slot = l % 2
            (_, bqkv, _, bo, ln1g, ln1b,
             _, b1, _, b2, ln2g, ln2b) = lrefs[12 * l:12 * (l + 1)]
            wait_layer(l)

            qkv = (jnp.dot(_bf(x), _bf(wq_buf[slot]),
                           preferred_element_type=jnp.float32) + bqkv[...])

            scores = []
            vhs = []
            for b in range(n_batch):
                rows = slice(b * S, (b + 1) * S)
                for h in range(H):
                    qh = qkv[rows, h * dh:(h + 1) * dh] * scale
                    kh = qkv[rows, D + h * dh:D + (h + 1) * dh]
                    vhs.append(qkv[rows, 2 * D + h * dh:2 * D + (h + 1) * dh])
                    scores.append(jax.lax.dot_general(
                        _bf(qh), _bf(kh), (((1,), (1,)), ((), ())),
                        preferred_element_type=jnp.float32))
            outs = []
            prev_p = None
            for i, s in enumerate(scores):
                p = jnp.exp(s - jnp.max(s, axis=-1, keepdims=True))
                p = p * pl.reciprocal(jnp.sum(p, axis=-1, keepdims=True),
                                      approx=True)
                if prev_p is not None:
                    outs.append(jnp.dot(_bf(prev_p), _bf(vhs[i - 1]),
                                        preferred_element_type=jnp.float32))
                prev_p = p
            outs.append(jnp.dot(_bf(prev_p), _bf(vhs[-1]),
                                preferred_element_type=jnp.float32))
            attn = jnp.concatenate(
                [jnp.concatenate(outs[b * H:(b + 1) * H], axis=-1)
                 for b in range(n_batch)], axis=0)

            proj = (jnp.dot(_bf(attn), _bf(wo_buf[slot]),
                            preferred_element_type=jnp.float32) + bo[...])
            x1 = _layernorm(proj + x, ln1g[...], ln1b[...])

            hdn = jnp.maximum(
                jnp.dot(_bf(x1), _bf(w1_buf[slot]),
                        preferred_element_type=jnp.float32) + b1[...], 0.0)
            y2 = (jnp.dot(_bf(hdn), _bf(w2_buf[slot]),
                          preferred_element_type=jnp.float32) + b2[...]) + x1
            x = _layernorm(y2, ln2g[...], ln2b[...])

            if l + 2 < _N_LAYERS:
                start_layer(l + 2)
            if l == _N_LAYERS - 2:
                mlm_copy.start()

        o_x_ref[...] = x.reshape(n_batch, S, D)
        mlm_copy.wait()
        logits = (jnp.dot(_bf(x), _bf(mbuf[...]),
                          preferred_element_type=jnp.float32) + mlm_b_ref[...])
        o_mlm_ref[...] = logits.reshape(n_batch, S, -1)

    return _body


def _encoder_and_mlm(x, layer_args, mlm_w, mlm_b, n_heads):
    B, S, D = x.shape
    d_fc = layer_args[6].shape[1]
    Npad = mlm_w.shape[1]
    n_prog = 2
    nb = B // n_prog

    _ANY = pl.BlockSpec(memory_space=pltpu.MemorySpace.HBM)

    def vspec(shape):
        nd = len(shape)
        return pl.BlockSpec(shape, lambda b: (0,) * nd)

    in_specs = [pl.BlockSpec((nb, S, D), lambda b: (b, 0, 0))]
    for i, a in enumerate(layer_args):
        in_specs.append(_ANY if i % 12 in (0, 2, 6, 8) else vspec(a.shape))
    in_specs += [_ANY, vspec(mlm_b.shape)]

    return pl.pallas_call(
        _make_fused_kernel(S, D, n_heads, d_fc, nb),
        out_shape=[jax.ShapeDtypeStruct((B, S, Npad), jnp.float32),
                   jax.ShapeDtypeStruct((B, S, D), jnp.float32)],
        grid=(n_prog,),
        in_specs=in_specs,
        out_specs=[pl.BlockSpec((nb, S, Npad), lambda b: (b, 0, 0)),
                   pl.BlockSpec((nb, S, D), lambda b: (b, 0, 0))],
        scratch_shapes=[
            pltpu.VMEM((2, D, 3 * D), jnp.float32),
            pltpu.VMEM((2, D, D), jnp.float32),
            pltpu.VMEM((2, D, d_fc), jnp.float32),
            pltpu.VMEM((2, d_fc, D), jnp.float32),
            pltpu.VMEM((D, Npad), jnp.float32),
            pltpu.SemaphoreType.DMA((4, 2)),
            pltpu.SemaphoreType.DMA,
        ],
        compiler_params=pltpu.CompilerParams(
            dimension_semantics=("parallel",)),
    )(x, *layer_args, mlm_w, mlm_b)


def kernel(x_ids, token_emb, pos_emb, mlm_w_pad, mlm_b_pad, cls_w, cls_b,
           l0_wqkv, l0_bqkv, l0_wo, l0_bo, l0_w1, l0_b1, l0_w2, l0_b2,
           l0_ln1_g, l0_ln1_b, l0_ln2_g, l0_ln2_b,
           l1_wqkv, l1_bqkv, l1_wo, l1_bo, l1_w1, l1_b1, l1_w2, l1_b2,
           l1_ln1_g, l1_ln1_b, l1_ln2_g, l1_ln2_b,
           l2_wqkv, l2_bqkv, l2_wo, l2_bo, l2_w1, l2_b1, l2_w2, l2_b2,
           l2_ln1_g, l2_ln1_b, l2_ln2_g, l2_ln2_b,
           l3_wqkv, l3_bqkv, l3_wo, l3_bo, l3_w1, l3_b1, l3_w2, l3_b2,
           l3_ln1_g, l3_ln1_b, l3_ln2_g, l3_ln2_b):
    B, S = x_ids.shape
    mlm_cls_num = token_emb.shape[0]

    x = jnp.take(token_emb, x_ids, axis=0) + pos_emb[:S][None, :, :]

    layer_args = [
        l0_wqkv, l0_bqkv, l0_wo, l0_bo, l0_ln1_g, l0_ln1_b,
        l0_w1, l0_b1, l0_w2, l0_b2, l0_ln2_g, l0_ln2_b,
        l1_wqkv, l1_bqkv, l1_wo, l1_bo, l1_ln1_g, l1_ln1_b,
        l1_w1, l1_b1, l1_w2, l1_b2, l1_ln2_g, l1_ln2_b,
        l2_wqkv, l2_bqkv, l2_wo, l2_bo, l2_ln1_g, l2_ln1_b,
        l2_w1, l2_b1, l2_w2, l2_b2, l2_ln2_g, l2_ln2_b,
        l3_wqkv, l3_bqkv, l3_wo, l3_bo, l3_ln1_g, l3_ln1_b,
        l3_w1, l3_b1, l3_w2, l3_b2, l3_ln2_g, l3_ln2_b,
    ]

    logits_pad, x_out = _encoder_and_mlm(
        x, layer_args, mlm_w_pad, mlm_b_pad, _N_HEADS)

    yp_mlm = logits_pad[:, :, :mlm_cls_num]
    yp_cls = jnp.dot(x_out[:, 0, :], cls_w) + cls_b
    return yp_mlm, yp_cls


# speedup vs baseline: 1.4810x; 1.0635x over previous
import math

import jax
import jax.numpy as jnp
from jax.experimental import pallas as pl
from jax.experimental.pallas import tpu as pltpu

_BF16 = jnp.bfloat16
_N_LAYERS = 4
_N_HEADS = 8


def _bf(a):
    return a.astype(_BF16)


def _layernorm(y, g, b):
    mu = jnp.mean(y, axis=-1, keepdims=True)
    var = jnp.mean((y - mu) ** 2, axis=-1, keepdims=True)
    return (y - mu) * jax.lax.rsqrt(var + 1e-5) * g + b


def _make_fused_kernel(S, D, H, d_fc, n_batch):
    dh = D // H
    scale = 1.0 / math.sqrt(dh)
    M = n_batch * S

    def _body(*refs):
        x_ref = refs[0]
        lrefs = refs[1:1 + 12 * _N_LAYERS]
        mlm_w_hbm = refs[1 + 12 * _N_LAYERS]
        mlm_b_ref = refs[2 + 12 * _N_LAYERS]
        o_mlm_ref = refs[3 + 12 * _N_LAYERS]
        o_x_ref = refs[4 + 12 * _N_LAYERS]
        (wq_buf, wo_buf, w1_buf, w2_buf, mbuf, sems, msem) = refs[
            5 + 12 * _N_LAYERS:]

        bufs = (wq_buf, wo_buf, w1_buf, w2_buf)

        def layer_copies(l):
            slot = l % 2
            lw = lrefs[12 * l:12 * (l + 1)]
            hbm = (lw[0], lw[2], lw[6], lw[8])
            return [pltpu.make_async_copy(hbm[t], bufs[t].at[slot],
                                          sems.at[t, slot])
                    for t in range(4)]

        def start_layer(l):
            for c in layer_copies(l):
                c.start()

        start_layer(0)
        start_layer(1)
        mlm_copy = pltpu.make_async_copy(mlm_w_hbm, mbuf, msem)

        x = x_ref[...].reshape(M, D)
        for l in range(_N_LAYERS):
            slot = l % 2
            (_, bqkv, _, bo, ln1g, ln1b,
             _, b1, _, b2, ln2g, ln2b) = lrefs[12 * l:12 * (l + 1)]
            c_qkv, c_wo, c_w1, c_w2 = layer_copies(l)
            c_qkv.wait()

            qkv = (jnp.dot(_bf(x), _bf(wq_buf[slot]),
                           preferred_element_type=jnp.float32) + bqkv[...])

            scores = []
            vhs = []
            for b in range(n_batch):
                rows = slice(b * S, (b + 1) * S)
                for h in range(H):
                    qh = qkv[rows, h * dh:(h + 1) * dh] * scale
                    kh = qkv[rows, D + h * dh:D + (h + 1) * dh]
                    vhs.append(qkv[rows, 2 * D + h * dh:2 * D + (h + 1) * dh])
                    scores.append(jax.lax.dot_general(
                        _bf(qh), _bf(kh), (((1,), (1,)), ((), ())),
                        preferred_element_type=jnp.float32))
            outs = []
            prev_p = None
            for i, s in enumerate(scores):
                p = jnp.exp(s - jnp.max(s, axis=-1, keepdims=True))
                p = p * pl.reciprocal(jnp.sum(p, axis=-1, keepdims=True),
                                      approx=True)
                if prev_p is not None:
                    outs.append(jnp.dot(_bf(prev_p), _bf(vhs[i - 1]),
                                        preferred_element_type=jnp.float32))
                prev_p = p
            outs.append(jnp.dot(_bf(prev_p), _bf(vhs[-1]),
                                preferred_element_type=jnp.float32))
            attn = jnp.concatenate(
                [jnp.concatenate(outs[b * H:(b + 1) * H], axis=-1)
                 for b in range(n_batch)], axis=0)

            c_wo.wait()
            proj = (jnp.dot(_bf(attn), _bf(wo_buf[slot]),
                            preferred_element_type=jnp.float32) + bo[...])
            x1 = _layernorm(proj + x, ln1g[...], ln1b[...])

            c_w1.wait()
            hdn = jnp.maximum(
                jnp.dot(_bf(x1), _bf(w1_buf[slot]),
                        preferred_element_type=jnp.float32) + b1[...], 0.0)
            c_w2.wait()
            y2 = (jnp.dot(_bf(hdn), _bf(w2_buf[slot]),
                          preferred_element_type=jnp.float32) + b2[...]) + x1
            x = _layernorm(y2, ln2g[...], ln2b[...])

            if l + 2 < _N_LAYERS:
                start_layer(l + 2)
            if l == _N_LAYERS - 2:
                mlm_copy.start()

        o_x_ref[...] = x.reshape(n_batch, S, D)
        mlm_copy.wait()
        logits = (jnp.dot(_bf(x), _bf(mbuf[...]),
                          preferred_element_type=jnp.float32) + mlm_b_ref[...])
        o_mlm_ref[...] = logits.reshape(n_batch, S, -1)

    return _body


def _encoder_and_mlm(x, layer_args, mlm_w, mlm_b, n_heads):
    B, S, D = x.shape
    d_fc = layer_args[6].shape[1]
    Npad = mlm_w.shape[1]
    n_prog = 2
    nb = B // n_prog

    _ANY = pl.BlockSpec(memory_space=pltpu.MemorySpace.HBM)

    def vspec(shape):
        nd = len(shape)
        return pl.BlockSpec(shape, lambda b: (0,) * nd)

    in_specs = [pl.BlockSpec((nb, S, D), lambda b: (b, 0, 0))]
    for i, a in enumerate(layer_args):
        in_specs.append(_ANY if i % 12 in (0, 2, 6, 8) else vspec(a.shape))
    in_specs += [_ANY, vspec(mlm_b.shape)]

    return pl.pallas_call(
        _make_fused_kernel(S, D, n_heads, d_fc, nb),
        out_shape=[jax.ShapeDtypeStruct((B, S, Npad), jnp.float32),
                   jax.ShapeDtypeStruct((B, S, D), jnp.float32)],
        grid=(n_prog,),
        in_specs=in_specs,
        out_specs=[pl.BlockSpec((nb, S, Npad), lambda b: (b, 0, 0)),
                   pl.BlockSpec((nb, S, D), lambda b: (b, 0, 0))],
        scratch_shapes=[
            pltpu.VMEM((2, D, 3 * D), jnp.float32),
            pltpu.VMEM((2, D, D), jnp.float32),
            pltpu.VMEM((2, D, d_fc), jnp.float32),
            pltpu.VMEM((2, d_fc, D), jnp.float32),
            pltpu.VMEM((D, Npad), jnp.float32),
            pltpu.SemaphoreType.DMA((4, 2)),
            pltpu.SemaphoreType.DMA,
        ],
        compiler_params=pltpu.CompilerParams(
            dimension_semantics=("parallel",)),
    )(x, *layer_args, mlm_w, mlm_b)


def kernel(x_ids, token_emb, pos_emb, mlm_w_pad, mlm_b_pad, cls_w, cls_b,
           l0_wqkv, l0_bqkv, l0_wo, l0_bo, l0_w1, l0_b1, l0_w2, l0_b2,
           l0_ln1_g, l0_ln1_b, l0_ln2_g, l0_ln2_b,
           l1_wqkv, l1_bqkv, l1_wo, l1_bo, l1_w1, l1_b1, l1_w2, l1_b2,
           l1_ln1_g, l1_ln1_b, l1_ln2_g, l1_ln2_b,
           l2_wqkv, l2_bqkv, l2_wo, l2_bo, l2_w1, l2_b1, l2_w2, l2_b2,
           l2_ln1_g, l2_ln1_b, l2_ln2_g, l2_ln2_b,
           l3_wqkv, l3_bqkv, l3_wo, l3_bo, l3_w1, l3_b1, l3_w2, l3_b2,
           l3_ln1_g, l3_ln1_b, l3_ln2_g, l3_ln2_b):
    B, S = x_ids.shape
    mlm_cls_num = token_emb.shape[0]

    x = jnp.take(token_emb, x_ids, axis=0) + pos_emb[:S][None, :, :]

    layer_args = [
        l0_wqkv, l0_bqkv, l0_wo, l0_bo, l0_ln1_g, l0_ln1_b,
        l0_w1, l0_b1, l0_w2, l0_b2, l0_ln2_g, l0_ln2_b,
        l1_wqkv, l1_bqkv, l1_wo, l1_bo, l1_ln1_g, l1_ln1_b,
        l1_w1, l1_b1, l1_w2, l1_b2, l1_ln2_g, l1_ln2_b,
        l2_wqkv, l2_bqkv, l2_wo, l2_bo, l2_ln1_g, l2_ln1_b,
        l2_w1, l2_b1, l2_w2, l2_b2, l2_ln2_g, l2_ln2_b,
        l3_wqkv, l3_bqkv, l3_wo, l3_bo, l3_ln1_g, l3_ln1_b,
        l3_w1, l3_b1, l3_w2, l3_b2, l3_ln2_g, l3_ln2_b,
    ]

    logits_pad, x_out = _encoder_and_mlm(
        x, layer_args, mlm_w_pad, mlm_b_pad, _N_HEADS)

    yp_mlm = logits_pad[:, :, :mlm_cls_num]
    yp_cls = jnp.dot(x_out[:, 0, :], cls_w) + cls_b
    return yp_mlm, yp_cls
